# baseline (speedup 1.0000x reference)
import jax
import jax.numpy as jnp
from jax import lax
from jax.experimental import pallas as pl
from jax.experimental.pallas import tpu as pltpu, tpu_sc as plsc

_ROWS = 8192
_D = 1024
_NC = 2
_NS = 16
_NW = _NC * _NS
_RPW = _ROWS // _NW
_C = 32
_NBUF = 2
_NCH = _RPW // _C


def _copy_body(W_hbm, out_hbm, buf, lsem, ssem):
    wid = lax.axis_index("s") * _NC + lax.axis_index("c")
    base = wid * _RPW

    def load(i, b):
        return pltpu.make_async_copy(
            W_hbm.at[pl.ds(base + i * _C, _C)], buf.at[b], lsem.at[b])

    def store(i, b):
        return pltpu.make_async_copy(
            buf.at[b], out_hbm.at[pl.ds(base + i * _C, _C)], ssem.at[b])

    load(0, 0).start()
    for i in range(_NCH):
        b = i % _NBUF
        if i + 1 < _NCH:
            nb = (i + 1) % _NBUF
            if i + 1 >= _NBUF:
                store(i + 1 - _NBUF, nb).wait()
            load(i + 1, nb).start()
        load(i, b).wait()
        store(i, b).start()
    for i in range(max(0, _NCH - _NBUF), _NCH):
        store(i, i % _NBUF).wait()


_SC_C = 512
_SC_NCH = (_ROWS // 2) // _SC_C


def _scs_body(W_hbm, out_hbm, spbuf, lsem, ssem):
    cid = lax.axis_index("c")
    base = cid * (_ROWS // 2)

    def load(i, b):
        return pltpu.make_async_copy(
            W_hbm.at[pl.ds(base + i * _SC_C, _SC_C)], spbuf.at[b], lsem.at[b])

    def store(i, b):
        return pltpu.make_async_copy(
            spbuf.at[b], out_hbm.at[pl.ds(base + i * _SC_C, _SC_C)],
            ssem.at[b])

    load(0, 0).start()
    for i in range(_SC_NCH):
        b = i % 2
        if i + 1 < _SC_NCH:
            nb = (i + 1) % 2
            if i + 1 >= 2:
                store(i - 1, nb).wait()
            load(i + 1, nb).start()
        load(i, b).wait()
        store(i, b).start()
    for i in range(_SC_NCH - 2, _SC_NCH):
        store(i, i % 2).wait()


@jax.jit
def _copy(W):
    mesh = plsc.ScalarSubcoreMesh(axis_name="c", num_cores=2)
    return pl.kernel(
        _scs_body,
        out_type=jax.ShapeDtypeStruct((_ROWS, _D), jnp.float32),
        mesh=mesh,
        scratch_types=[
            pltpu.VMEM_SHARED((2, _SC_C, _D), jnp.float32),
            pltpu.SemaphoreType.DMA((2,)),
            pltpu.SemaphoreType.DMA((2,)),
        ],
    )(W)


def kernel(x, W):
    del x
    return _copy(W)[None]


# speedup vs baseline: 1.1449x; 1.1449x over previous
import jax
import jax.numpy as jnp
from jax import lax
from jax.experimental import pallas as pl
from jax.experimental.pallas import tpu as pltpu, tpu_sc as plsc
from jax._src.pallas import mpmd as _mpmd

_ROWS = 8192
_D = 1024
_NC = 2
_NS = 16
_NW = _NC * _NS

_TEC_ROWS = 4096
_RPW = _TEC_ROWS // _NW
_C = 32
_NCH = _RPW // _C

_SCS_ROWS = _ROWS - _TEC_ROWS
_SC_C = 512
_SC_NCH = (_SCS_ROWS // _NC) // _SC_C


def _tec_fn(W_hbm, out_hbm, spbuf):
    del spbuf

    def body(buf, lsem, ssem):
        wid = lax.axis_index("s") * _NC + lax.axis_index("c")
        base = wid * _RPW

        def load(i, b):
            return pltpu.make_async_copy(
                W_hbm.at[pl.ds(base + i * _C, _C)], buf.at[b], lsem.at[b])

        def store(i, b):
            return pltpu.make_async_copy(
                buf.at[b], out_hbm.at[pl.ds(base + i * _C, _C)], ssem.at[b])

        load(0, 0).start()
        for i in range(_NCH):
            b = i % 2
            if i + 1 < _NCH:
                nb = (i + 1) % 2
                if i + 1 >= 2:
                    store(i - 1, nb).wait()
                load(i + 1, nb).start()
            load(i, b).wait()
            store(i, b).start()
        for i in range(max(0, _NCH - 2), _NCH):
            store(i, i % 2).wait()

    pl.run_scoped(
        body,
        pltpu.VMEM((2, _C, _D), jnp.float32),
        pltpu.SemaphoreType.DMA((2,)),
        pltpu.SemaphoreType.DMA((2,)),
    )


def _scs_fn(W_hbm, out_hbm, spbuf):
    def body(lsem, ssem):
        cid = lax.axis_index("c")
        base = _TEC_ROWS + cid * (_SCS_ROWS // _NC)

        def load(i, b):
            return pltpu.make_async_copy(
                W_hbm.at[pl.ds(base + i * _SC_C, _SC_C)], spbuf.at[b],
                lsem.at[b])

        def store(i, b):
            return pltpu.make_async_copy(
                spbuf.at[b], out_hbm.at[pl.ds(base + i * _SC_C, _SC_C)],
                ssem.at[b])

        load(0, 0).start()
        for i in range(_SC_NCH):
            b = i % 2
            if i + 1 < _SC_NCH:
                nb = (i + 1) % 2
                if i + 1 >= 2:
                    store(i - 1, nb).wait()
                load(i + 1, nb).start()
            load(i, b).wait()
            store(i, b).start()
        for i in range(max(0, _SC_NCH - 2), _SC_NCH):
            store(i, i % 2).wait()

    pl.run_scoped(
        body,
        pltpu.SemaphoreType.DMA((2,)),
        pltpu.SemaphoreType.DMA((2,)),
    )


@jax.jit
def _copy(W):
    scalar_mesh = plsc.ScalarSubcoreMesh(axis_name="c", num_cores=_NC)
    vector_mesh = plsc.VectorSubcoreMesh(
        core_axis_name="c", subcore_axis_name="s")
    return _mpmd.mpmd_map(
        [(vector_mesh, _tec_fn), (scalar_mesh, _scs_fn)],
        jax.ShapeDtypeStruct((_ROWS, _D), jnp.float32),
        scratch_types=(
            pltpu.VMEM_SHARED((2, _SC_C, _D), jnp.float32),
        ),
    )(W)


def kernel(x, W):
    del x
    return _copy(W)[None]


# speedup vs baseline: 1.1471x; 1.0020x over previous
import jax
import jax.numpy as jnp
from jax import lax
from jax.experimental import pallas as pl
from jax.experimental.pallas import tpu as pltpu, tpu_sc as plsc
from jax._src.pallas import mpmd as _mpmd

_ROWS = 8192
_D = 1024
_NC = 2
_NS = 16
_NW = _NC * _NS

_TEC_ROWS = 5120
_RPW = _TEC_ROWS // _NW
_C = 32
_NCH = _RPW // _C

_SCS_ROWS = _ROWS - _TEC_ROWS
_SC_C = 512
_SC_NCH = (_SCS_ROWS // _NC) // _SC_C


def _tec_fn(W_hbm, out_hbm, spbuf):
    del spbuf

    def body(buf, lsem, ssem):
        wid = lax.axis_index("s") * _NC + lax.axis_index("c")
        base = wid * _RPW

        def load(i, b):
            return pltpu.make_async_copy(
                W_hbm.at[pl.ds(base + i * _C, _C)], buf.at[b], lsem.at[b])

        def store(i, b):
            return pltpu.make_async_copy(
                buf.at[b], out_hbm.at[pl.ds(base + i * _C, _C)], ssem.at[b])

        load(0, 0).start()
        for i in range(_NCH):
            b = i % 2
            if i + 1 < _NCH:
                nb = (i + 1) % 2
                if i + 1 >= 2:
                    store(i - 1, nb).wait()
                load(i + 1, nb).start()
            load(i, b).wait()
            store(i, b).start()
        for i in range(max(0, _NCH - 2), _NCH):
            store(i, i % 2).wait()

    pl.run_scoped(
        body,
        pltpu.VMEM((2, _C, _D), jnp.float32),
        pltpu.SemaphoreType.DMA((2,)),
        pltpu.SemaphoreType.DMA((2,)),
    )


def _scs_fn(W_hbm, out_hbm, spbuf):
    def body(lsem, ssem):
        cid = lax.axis_index("c")
        base = _TEC_ROWS + cid * (_SCS_ROWS // _NC)

        def load(i, b):
            return pltpu.make_async_copy(
                W_hbm.at[pl.ds(base + i * _SC_C, _SC_C)], spbuf.at[b],
                lsem.at[b])

        def store(i, b):
            return pltpu.make_async_copy(
                spbuf.at[b], out_hbm.at[pl.ds(base + i * _SC_C, _SC_C)],
                ssem.at[b])

        load(0, 0).start()
        for i in range(_SC_NCH):
            b = i % 2
            if i + 1 < _SC_NCH:
                nb = (i + 1) % 2
                if i + 1 >= 2:
                    store(i - 1, nb).wait()
                load(i + 1, nb).start()
            load(i, b).wait()
            store(i, b).start()
        for i in range(max(0, _SC_NCH - 2), _SC_NCH):
            store(i, i % 2).wait()

    pl.run_scoped(
        body,
        pltpu.SemaphoreType.DMA((2,)),
        pltpu.SemaphoreType.DMA((2,)),
    )


@jax.jit
def _copy(W):
    scalar_mesh = plsc.ScalarSubcoreMesh(axis_name="c", num_cores=_NC)
    vector_mesh = plsc.VectorSubcoreMesh(
        core_axis_name="c", subcore_axis_name="s")
    return _mpmd.mpmd_map(
        [(vector_mesh, _tec_fn), (scalar_mesh, _scs_fn)],
        jax.ShapeDtypeStruct((_ROWS, _D), jnp.float32),
        scratch_types=(
            pltpu.VMEM_SHARED((2, _SC_C, _D), jnp.float32),
        ),
    )(W)


def kernel(x, W):
    del x
    return _copy(W)[None]
